# baseline (speedup 1.0000x reference)
import functools

import jax
import jax.numpy as jnp
import numpy as np
from jax.experimental import pallas as pl
from jax.experimental.pallas import tpu as pltpu

_MAX_SPIKE = 100.0
_P = 0.1
_NCHUNK = 32
_W = 128

_cache = {}


def _spike_consts(B, T):
    if (B, T) not in _cache:
        with jax.ensure_compile_time_eval():
            key = jax.random.key(42)
            k1, k2, k3 = jax.random.split(key, 3)
            probas = jax.random.uniform(k1, (B,), dtype=jnp.float32)
            mask = probas > (1.0 - _P)
            pos = jax.random.randint(k2, (B,), 0, T - 2)
            sign = jnp.where(jax.random.randint(k3, (), 0, 2) == 0, -1.0, 1.0)
            mask_np = np.asarray(jax.device_get(mask))
            pos_np = np.asarray(jax.device_get(pos))
            sign_f = float(jax.device_get(sign))
        rows = [int(i) for i in np.nonzero(mask_np)[0]]
        spiked = set(rows)
        dummy = next(i for i in range(B) if i not in spiked)
        k = len(rows)
        kpad = max(8, -(-k // 8) * 8)
        triples = []
        for i in rows:
            p = int(pos_np[i])
            w0 = (p // _W) * _W
            patch = np.zeros((_W,), np.float32)
            patch[p - w0] = sign_f * _MAX_SPIKE
            triples.append((i, w0, patch))
        for _ in range(kpad - k):
            triples.append((dummy, 0, np.zeros((_W,), np.float32)))
        patch_np = np.stack([t[2] for t in triples])
        _cache[(B, T)] = (
            [(t[0], t[1]) for t in triples], patch_np)
    return _cache[(B, T)]


def _body(windows, B, T, x_hbm, patch_ref, o_hbm, win, sem_bulk, sem_win):
    rows_per_chunk = B // _NCHUNK
    bulk = []
    for c in range(_NCHUNK):
        r0 = c * rows_per_chunk
        bulk.append(pltpu.async_copy(x_hbm.at[pl.ds(r0, rows_per_chunk)],
                                     o_hbm.at[pl.ds(r0, rows_per_chunk)],
                                     sem_bulk))
    gh = [pltpu.async_copy(x_hbm.at[pl.ds(r, 1), pl.ds(w0, _W)],
                           win.at[pl.ds(t, 1)], sem_win)
          for t, (r, w0) in enumerate(windows)]
    for h in gh:
        h.wait()
    win[...] = win[...] + patch_ref[...]
    for h in bulk:
        h.wait()
    sh = [pltpu.async_copy(win.at[pl.ds(t, 1)],
                           o_hbm.at[pl.ds(r, 1), pl.ds(w0, _W)], sem_win)
          for t, (r, w0) in enumerate(windows)]
    for h in sh:
        h.wait()


def kernel(x):
    B, T = x.shape
    windows, patch_np = _spike_consts(B, T)
    kpad = patch_np.shape[0]
    return pl.pallas_call(
        functools.partial(_body, windows, B, T),
        in_specs=[
            pl.BlockSpec(memory_space=pltpu.HBM),
            pl.BlockSpec(memory_space=pltpu.VMEM),
        ],
        out_specs=pl.BlockSpec(memory_space=pltpu.HBM),
        out_shape=jax.ShapeDtypeStruct((B, T), x.dtype),
        scratch_shapes=[
            pltpu.VMEM((kpad, _W), jnp.float32),
            pltpu.SemaphoreType.DMA,
            pltpu.SemaphoreType.DMA,
        ],
    )(x, jnp.asarray(patch_np))


# speedup vs baseline: 34.6767x; 34.6767x over previous
import jax
import jax.numpy as jnp
from jax.experimental import pallas as pl

_MAX_SPIKE = 100.0
_P = 0.1


def _spike_consts(B, T, dtype):
    key = jax.random.key(42)
    k1, k2, k3 = jax.random.split(key, 3)
    probas = jax.random.uniform(k1, (B,), dtype=jnp.float32)
    mask = probas > (1.0 - _P)
    pos = jax.random.randint(k2, (B,), 0, T - 2)
    sign = jnp.where(jax.random.randint(k3, (), 0, 2) == 0, -1.0, 1.0).astype(dtype)
    vals = jnp.where(mask, sign * _MAX_SPIKE, 0.0).astype(dtype)
    return pos, vals


def _body(pos_ref, val_ref, x_ref, o_ref):
    j = pl.program_id(1)
    bc = x_ref.shape[-1]
    cols = jax.lax.broadcasted_iota(jnp.int32, x_ref.shape, 1) + j * bc
    o_ref[...] = x_ref[...] + jnp.where(cols == pos_ref[...], val_ref[...], 0.0)


def kernel(x):
    B, T = x.shape
    pos, vals = _spike_consts(B, T, x.dtype)
    BR, BC = 128, 8192
    grid = (B // BR, T // BC)
    return pl.pallas_call(
        _body,
        grid=grid,
        in_specs=[
            pl.BlockSpec((BR, 1), lambda i, j: (i, 0)),
            pl.BlockSpec((BR, 1), lambda i, j: (i, 0)),
            pl.BlockSpec((BR, BC), lambda i, j: (i, j)),
        ],
        out_specs=pl.BlockSpec((BR, BC), lambda i, j: (i, j)),
        out_shape=jax.ShapeDtypeStruct((B, T), x.dtype),
    )(pos[:, None], vals[:, None], x)


# speedup vs baseline: 35.0597x; 1.0110x over previous
import jax
import jax.numpy as jnp
from jax.experimental import pallas as pl

_MAX_SPIKE = 100.0
_P = 0.1


def _spike_consts(B, T, dtype):
    key = jax.random.key(42)
    k1, k2, k3 = jax.random.split(key, 3)
    probas = jax.random.uniform(k1, (B,), dtype=jnp.float32)
    mask = probas > (1.0 - _P)
    pos = jax.random.randint(k2, (B,), 0, T - 2)
    sign = jnp.where(jax.random.randint(k3, (), 0, 2) == 0, -1.0, 1.0).astype(dtype)
    vals = jnp.where(mask, sign * _MAX_SPIKE, 0.0).astype(dtype)
    return pos, vals


def _body(pos_ref, val_ref, x_ref, o_ref):
    j = pl.program_id(1)
    bc = x_ref.shape[-1]
    cols = jax.lax.broadcasted_iota(jnp.int32, x_ref.shape, 1) + j * bc
    o_ref[...] = x_ref[...] + jnp.where(cols == pos_ref[...], val_ref[...], 0.0)


def kernel(x):
    B, T = x.shape
    pos, vals = _spike_consts(B, T, x.dtype)
    BR, BC = 256, 8192
    grid = (B // BR, T // BC)
    return pl.pallas_call(
        _body,
        grid=grid,
        in_specs=[
            pl.BlockSpec((BR, 1), lambda i, j: (i, 0)),
            pl.BlockSpec((BR, 1), lambda i, j: (i, 0)),
            pl.BlockSpec((BR, BC), lambda i, j: (i, j)),
        ],
        out_specs=pl.BlockSpec((BR, BC), lambda i, j: (i, j)),
        out_shape=jax.ShapeDtypeStruct((B, T), x.dtype),
    )(pos[:, None], vals[:, None], x)


# speedup vs baseline: 35.1644x; 1.0030x over previous
import jax
import jax.numpy as jnp
from jax.experimental import pallas as pl

_MAX_SPIKE = 100.0
_P = 0.1


def _spike_consts(B, T, dtype):
    key = jax.random.key(42)
    k1, k2, k3 = jax.random.split(key, 3)
    probas = jax.random.uniform(k1, (B,), dtype=jnp.float32)
    mask = probas > (1.0 - _P)
    pos = jax.random.randint(k2, (B,), 0, T - 2)
    sign = jnp.where(jax.random.randint(k3, (), 0, 2) == 0, -1.0, 1.0).astype(dtype)
    vals = jnp.where(mask, sign * _MAX_SPIKE, 0.0).astype(dtype)
    return pos, vals


def _body(pos_ref, val_ref, x_ref, o_ref):
    j = pl.program_id(1)
    bc = x_ref.shape[-1]
    cols = jax.lax.broadcasted_iota(jnp.int32, x_ref.shape, 1) + j * bc
    o_ref[...] = x_ref[...] + jnp.where(cols == pos_ref[...], val_ref[...], 0.0)


def kernel(x):
    B, T = x.shape
    pos, vals = _spike_consts(B, T, x.dtype)
    BR, BC = 64, 32768
    grid = (B // BR, T // BC)
    return pl.pallas_call(
        _body,
        grid=grid,
        in_specs=[
            pl.BlockSpec((BR, 1), lambda i, j: (i, 0)),
            pl.BlockSpec((BR, 1), lambda i, j: (i, 0)),
            pl.BlockSpec((BR, BC), lambda i, j: (i, j)),
        ],
        out_specs=pl.BlockSpec((BR, BC), lambda i, j: (i, j)),
        out_shape=jax.ShapeDtypeStruct((B, T), x.dtype),
    )(pos[:, None], vals[:, None], x)
